# Initial kernel scaffold; baseline (speedup 1.0000x reference)
import jax
import jax.numpy as jnp
from jax.experimental import pallas as pl


def kernel(node_feats1, node_feats2, node_feats3, edge_feats1, edge_feats2, edge_feats3, edge_index1, edge_index2, edge_index3, node_graph_ids1, node_graph_ids2, node_graph_ids3, params1, params2, params3):
    raise NotImplementedError("write your pallas kernel here")


# speedup vs baseline: 4.7612x; 4.7612x over previous
import functools

import jax
import jax.numpy as jnp
from jax import lax
from jax.experimental import pallas as pl
from jax.experimental.pallas import tpu as pltpu, tpu_sc as plsc

N = 10000
E = 160000
G = 200
DP = 256
DQ = 128
DH = 128
NACC = 10240
NG = 64
BLKN = 1000
BLKE = 6400
NC, NS = 2, 16
NW = NC * NS
CH = 128

f32 = jnp.float32


def _lrelu(x):
    return jnp.where(x > 0, x, 0.01 * x)


def _elu(x):
    return jnp.where(x > 0, x, jnp.exp(x) - 1.0)


@functools.lru_cache(maxsize=None)
def _sc_gather_kernel(D, B):
    n_chunks = B // CH
    n_iter = (n_chunks + NW - 1) // NW
    mesh = plsc.VectorSubcoreMesh(core_axis_name="c", subcore_axis_name="s")

    @functools.partial(
        pl.kernel, mesh=mesh,
        out_type=jax.ShapeDtypeStruct((B, D), f32),
        scratch_types=[
            pltpu.VMEM((CH,), jnp.int32),
            pltpu.VMEM((CH, D), f32),
            pltpu.SemaphoreType.DMA,
        ],
    )
    def k(table_hbm, idx_hbm, out_hbm, idx_v, rows_v, sem):
        c = lax.axis_index("c")
        s = lax.axis_index("s")
        wid = s * NC + c

        def body(i, carry):
            cid = i * NW + wid

            @pl.when(cid < n_chunks)
            def _():
                base = cid * CH
                pltpu.sync_copy(idx_hbm.at[pl.ds(base, CH)], idx_v)
                pltpu.async_copy(table_hbm.at[idx_v], rows_v, sem).wait()
                pltpu.sync_copy(rows_v, out_hbm.at[pl.ds(base, CH)])
            return carry

        lax.fori_loop(0, n_iter, body, 0)

    return k


CHS = 640
CPT = 8


@functools.lru_cache(maxsize=None)
def _sc_scatter_kernel(B):
    n_chunks = B // CHS
    mesh = plsc.VectorSubcoreMesh(core_axis_name="c", subcore_axis_name="s")

    @functools.partial(
        pl.kernel, mesh=mesh,
        compiler_params=pltpu.CompilerParams(needs_layout_passes=False),
        out_type=jax.ShapeDtypeStruct((DP * NACC,), f32),
        scratch_types=[
            pltpu.VMEM((2 * CHS,), jnp.int32),
            pltpu.VMEM((2, CPT, CHS), f32),
            pltpu.VMEM((CPT * NACC,), f32),
            pltpu.SemaphoreType.DMA,
            pltpu.SemaphoreType.DMA,
            pltpu.SemaphoreType.DMA,
            pltpu.SemaphoreType.DMA,
        ],
    )
    def k(rowsT_hbm, idx_hbm, zerosT_hbm, out_hbm, idx_v, buf_v, acc_v,
          si0, si1, sr0, sr1):
        c = lax.axis_index("c")
        s = lax.axis_index("s")
        wid = s * NC + c
        rbase = wid * CPT
        sis = (si0, si1)
        srs = (sr0, sr1)
        pltpu.sync_copy(zerosT_hbm, acc_v)

        def start(cid, b):
            pltpu.async_copy(idx_hbm.at[pl.ds(cid * CHS, CHS)],
                             idx_v.at[pl.ds(b * CHS, CHS)], sis[b])
            pltpu.async_copy(
                rowsT_hbm.at[pl.ds(rbase, CPT), pl.ds(cid * CHS, CHS)],
                buf_v.at[b], srs[b])

        start(0, 0)
        start(1, 1)

        def body(i, carry):
            for b in range(2):
                cid = i * 2 + b
                pltpu.make_async_copy(idx_hbm.at[pl.ds(cid * CHS, CHS)],
                                      idx_v.at[pl.ds(b * CHS, CHS)],
                                      sis[b]).wait()
                pltpu.make_async_copy(
                    rowsT_hbm.at[pl.ds(rbase, CPT), pl.ds(cid * CHS, CHS)],
                    buf_v.at[b], srs[b]).wait()
                def grp(g, cc):
                    dstv = idx_v[pl.ds(b * CHS + g * 16, 16)]
                    for j in range(CPT):
                        vals = buf_v[b, j, pl.ds(g * 16, 16)]
                        plsc.addupdate_scatter(acc_v, [dstv + (j * NACC)], vals)
                    return cc

                lax.fori_loop(0, CHS // 16, grp, 0)

                @pl.when(cid + 2 < n_chunks)
                def _():
                    start(cid + 2, b)
            return carry

        lax.fori_loop(0, n_chunks // 2, body, 0)
        pltpu.sync_copy(acc_v, out_hbm.at[pl.ds(rbase * NACC, CPT * NACC)])

    return k


def _gather_rows(table, idx, D):
    return _sc_gather_kernel(D, idx.shape[0])(table, idx)


def _scatter_add_rows(rowsT, idx, zerosT):
    accT = _sc_scatter_kernel(rowsT.shape[1])(rowsT, idx, zerosT)
    return accT.reshape(DP, NACC)[:, :N].T


def _mm(x, w):
    return jax.lax.dot_general(x, w, (((1,), (0,)), ((), ())),
                               preferred_element_type=f32)


def _mmT0(x, y):
    return jax.lax.dot_general(x, y, (((0,), (0,)), ((), ())),
                               preferred_element_type=f32)


def _gru_body(x, h, w):
    r = jax.nn.sigmoid(_mm(x, w['WrT']) + w['bir'] + _mm(h, w['UrT']) + w['bhr'])
    z = jax.nn.sigmoid(_mm(x, w['WzT']) + w['biz'] + _mm(h, w['UzT']) + w['bhz'])
    n = jnp.tanh(_mm(x, w['WnT']) + w['bin'] + r * (_mm(h, w['UnT']) + w['bhn']))
    return (1.0 - z) * n + z * h


_GRU_KEYS = ('WrT', 'WzT', 'WnT', 'UrT', 'UzT', 'UnT',
             'bir', 'biz', 'bin', 'bhr', 'bhz', 'bhn')


def _gru_prep(Wih, Whh, bih, bhh):
    return {
        'WrT': Wih[0:G].T, 'WzT': Wih[G:2 * G].T, 'WnT': Wih[2 * G:].T,
        'UrT': Whh[0:G].T, 'UzT': Whh[G:2 * G].T, 'UnT': Whh[2 * G:].T,
        'bir': bih[0:G][None, :], 'biz': bih[G:2 * G][None, :],
        'bin': bih[2 * G:][None, :],
        'bhr': bhh[0:G][None, :], 'bhz': bhh[G:2 * G][None, :],
        'bhn': bhh[2 * G:][None, :],
    }


def _row_spec(blk, w):
    return pl.BlockSpec((blk, w), lambda i: (i, 0))


def _full_spec(shape):
    nd = len(shape)
    return pl.BlockSpec(shape, lambda *_: (0,) * nd)


def _t0a_body(nf, pnWT, pnb, ae1T, Wq, bq, hv_o, tsrc_o, tdst_o):
    hv = _lrelu(_mm(nf[...], pnWT[...]) + pnb[...])
    hv_o[...] = hv
    tsrc_o[...] = _mm(nf[...], ae1T[...])
    tdst_o[...] = _mm(hv, Wq[...]) + bq[...]


def _t0a(nf, pnWT, pnb, ae1T, Wq, bq):
    grid = (N // BLKN,)
    return pl.pallas_call(
        _t0a_body,
        grid=grid,
        in_specs=[_row_spec(BLKN, 128), _full_spec((128, G)), _full_spec((1, G)),
                  _full_spec((128, DP)), _full_spec((G, DQ)), _full_spec((1, DQ))],
        out_specs=[_row_spec(BLKN, G), _row_spec(BLKN, DP), _row_spec(BLKN, DQ)],
        out_shape=[jax.ShapeDtypeStruct((N, G), f32),
                   jax.ShapeDtypeStruct((N, DP), f32),
                   jax.ShapeDtypeStruct((N, DQ), f32)],
    )(nf, pnWT, pnb, ae1T, Wq, bq)


def _t0b_body(ef, WeT, b1p, out):
    out[...] = _mm(ef[...], WeT[...]) + b1p[...]


def _t0b(ef, WeT, b1p):
    return pl.pallas_call(
        _t0b_body,
        grid=(E // BLKE,),
        in_specs=[_row_spec(BLKE, 16), _full_spec((16, DP)), _full_spec((1, DP))],
        out_specs=_row_spec(BLKE, DP),
        out_shape=jax.ShapeDtypeStruct((E, DP), f32),
    )(ef, WeT, b1p)


def _te1_body(gs, b1, gd, vp, out):
    he1 = _lrelu(gs[...] + b1[...])
    dot = jnp.sum(he1 * vp[...], axis=1, keepdims=True)
    ex = jnp.exp(_lrelu(gd[..., 0:1] + dot))
    col = lax.broadcasted_iota(jnp.int32, he1.shape, 1)
    out[...] = jnp.where(col == G, ex, he1 * ex).T


def _te1(gs, b1, gd, vp):
    return pl.pallas_call(
        _te1_body,
        grid=(E // BLKE,),
        in_specs=[_row_spec(BLKE, DP), _row_spec(BLKE, DP),
                  _row_spec(BLKE, DQ), _full_spec((1, DP))],
        out_specs=pl.BlockSpec((DP, BLKE), lambda i: (0, i)),
        out_shape=jax.ShapeDtypeStruct((DP, E), f32),
    )(gs, b1, gd, vp)


def _te2_body(gs, gd, out):
    g = gs[...]
    ex = jnp.exp(_lrelu(g[..., G:G + 1] + gd[..., 0:1]))
    col = lax.broadcasted_iota(jnp.int32, g.shape, 1)
    out[...] = jnp.where(col == G, ex, g * ex).T


def _te2(gs, gd):
    return pl.pallas_call(
        _te2_body,
        grid=(E // BLKE,),
        in_specs=[_row_spec(BLKE, DP), _row_spec(BLKE, DQ)],
        out_specs=pl.BlockSpec((DP, BLKE), lambda i: (0, i)),
        out_shape=jax.ShapeDtypeStruct((DP, E), f32),
    )(gs, gd)


def _acc_combine(a):
    s0 = a[..., G:G + 1]
    safe = jnp.where(s0 > 0, s0, 1.0)
    return s0, a[..., 0:G] / safe


def _tn1_body(a0, hv, etWT, etb, M1, c1, Wq, bq, *gw_refs):
    w = {k: r[...] for k, r in zip(_GRU_KEYS, gw_refs[:12])}
    s0, mean = _acc_combine(a0[...])
    ctx = _elu(jnp.where(s0 > 0, _mm(mean, etWT[...]) + etb[...], 0.0))
    node = jax.nn.relu(_gru_body(ctx, hv[...], w))
    gw_refs[12][...] = node
    gw_refs[13][...] = _mm(node, M1[...]) + c1[...]
    gw_refs[14][...] = _mm(node, Wq[...]) + bq[...]


def _tn1(a0, hv, etWT, etb, M1, c1, Wq, bq, gw):
    ins = [a0, hv, etWT, etb, M1, c1, Wq, bq] + [gw[k] for k in _GRU_KEYS]
    in_specs = ([_row_spec(BLKN, DP), _row_spec(BLKN, G),
                 _full_spec((G, G)), _full_spec((1, G)),
                 _full_spec((G, DP)), _full_spec((1, DP)),
                 _full_spec((G, DQ)), _full_spec((1, DQ))]
                + [_full_spec(gw[k].shape) for k in _GRU_KEYS])
    return pl.pallas_call(
        _tn1_body,
        grid=(N // BLKN,),
        in_specs=in_specs,
        out_specs=[_row_spec(BLKN, G), _row_spec(BLKN, DP), _row_spec(BLKN, DQ)],
        out_shape=[jax.ShapeDtypeStruct((N, G), f32),
                   jax.ShapeDtypeStruct((N, DP), f32),
                   jax.ShapeDtypeStruct((N, DQ), f32)],
    )(*ins)


def _tn2a_body(a0, hprev, M1, c1, Wq, bq, *gw_refs):
    w = {k: r[...] for k, r in zip(_GRU_KEYS, gw_refs[:12])}
    s0, mean = _acc_combine(a0[...])
    cin = _elu(jnp.where(s0 > 0, mean, 0.0))
    node = jax.nn.relu(_gru_body(cin, hprev[...], w))
    gw_refs[12][...] = node
    gw_refs[13][...] = _mm(node, M1[...]) + c1[...]
    gw_refs[14][...] = _mm(node, Wq[...]) + bq[...]


def _tn2a(a0, hprev, M1, c1, Wq, bq, gw):
    ins = [a0, hprev, M1, c1, Wq, bq] + [gw[k] for k in _GRU_KEYS]
    in_specs = ([_row_spec(BLKN, DP), _row_spec(BLKN, G),
                 _full_spec((G, DP)), _full_spec((1, DP)),
                 _full_spec((G, DQ)), _full_spec((1, DQ))]
                + [_full_spec(gw[k].shape) for k in _GRU_KEYS])
    return pl.pallas_call(
        _tn2a_body,
        grid=(N // BLKN,),
        in_specs=in_specs,
        out_specs=[_row_spec(BLKN, G), _row_spec(BLKN, DP), _row_spec(BLKN, DQ)],
        out_shape=[jax.ShapeDtypeStruct((N, G), f32),
                   jax.ShapeDtypeStruct((N, DP), f32),
                   jax.ShapeDtypeStruct((N, DQ), f32)],
    )(*ins)


def _tn2b_body(a0, hprev, *gw_refs):
    w = {k: r[...] for k, r in zip(_GRU_KEYS, gw_refs[:12])}
    s0, mean = _acc_combine(a0[...])
    cin = _elu(jnp.where(s0 > 0, mean, 0.0))
    gw_refs[12][...] = jax.nn.relu(_gru_body(cin, hprev[...], w))


def _tn2b(a0, hprev, gw):
    ins = [a0, hprev] + [gw[k] for k in _GRU_KEYS]
    in_specs = ([_row_spec(BLKN, DP), _row_spec(BLKN, G)]
                + [_full_spec(gw[k].shape) for k in _GRU_KEYS])
    return pl.pallas_call(
        _tn2b_body,
        grid=(N // BLKN,),
        in_specs=in_specs,
        out_specs=_row_spec(BLKN, G),
        out_shape=jax.ShapeDtypeStruct((N, G), f32),
    )(*ins)


def _tr_body(node_r, gid_r, predWT, predb, *prefs):
    node = node_r[...]
    gid = gid_r[...]
    oh = (gid == lax.broadcasted_iota(jnp.int32, (N, NG), 1)).astype(f32)
    g = _mmT0(oh, node)
    for t in range(2):
        base = t * 17
        R1mat, clb, Zw, prnWT, prnb = [r[...] for r in prefs[base:base + 5]]
        w = {k: r[...] for k, r in zip(_GRU_KEYS, prefs[base + 5:base + 17])}
        r1 = _mm(jax.nn.relu(g), R1mat) + clb
        zn = _mm(node, Zw)
        z = _lrelu(_mm(oh, r1) + zn)
        ex = jnp.exp(z - jnp.max(z))
        exv = ex[..., 0:1]
        s = _mmT0(oh, exv)
        hv = _mm(node, prnWT) + prnb
        num = _mmT0(oh, exv * hv)
        grepr = jnp.where(s > 0, num / jnp.where(s > 0, s, 1.0), 0.0)
        g = jax.nn.relu(_gru_body(_elu(grepr), g, w))
    prefs[34][...] = _mm(g, predWT[...]) + predb[...]


def _tr(node, gid2d, predWT, predb, pflat):
    ins = [node, gid2d, predWT, predb] + pflat
    in_specs = [_full_spec(x.shape) for x in ins]
    return pl.pallas_call(
        _tr_body,
        in_specs=in_specs,
        out_specs=_full_spec((NG, 128)),
        out_shape=jax.ShapeDtypeStruct((NG, 128), f32),
    )(*ins)


def _pad_cols(x, w):
    return jnp.pad(x, ((0, 0), (0, w - x.shape[1])))


def _qcol(vec, bias):
    Wq = jnp.zeros((G, DQ), f32).at[:, 0].set(vec)
    bq = jnp.zeros((1, DQ), f32).at[0, 0].set(bias)
    return Wq, bq


def _tower(p, nf, ef, ei, gid, zeros):
    src = ei[0]
    dst = ei[1]

    pnWT = p['pn_W'].T
    pnb = p['pn_b'][None, :]
    ae1T = _pad_cols(p['pe1_W'][:, :128].T, DP)
    Wq1, bq1 = _qcol(p['pe2_W'][0, :G], p['pe2_b'][0])
    WeT = _pad_cols(p['pe1_W'][:, 128:].T, DP)
    b1p = _pad_cols(p['pe1_b'][None, :], DP)
    vp = _pad_cols(p['pe2_W'][0, G:][None, :], DP)
    gw1 = _gru_prep(p['ag1_Wih'], p['ag1_Whh'], p['ag1_bih'], p['ag1_bhh'])

    def layer_tabs(lp):
        M = jnp.concatenate(
            [lp['pn_W'].T, lp['pe_W'][0, G:][:, None],
             jnp.zeros((G, DP - G - 1), f32)], axis=1)
        c = jnp.concatenate(
            [lp['pn_b'], jnp.zeros((DP - G,), f32)])[None, :]
        Wq, bq = _qcol(lp['pe_W'][0, :G], lp['pe_b'][0])
        return M, c, Wq, bq

    M1, c1, Wq_l0, bq_l0 = layer_tabs(p['layers'][0])
    M2, c2, Wq_l1, bq_l1 = layer_tabs(p['layers'][1])
    gw_l0 = _gru_prep(*[p['layers'][0][k] for k in ('Wih', 'Whh', 'bih', 'bhh')])
    gw_l1 = _gru_prep(*[p['layers'][1][k] for k in ('Wih', 'Whh', 'bih', 'bhh')])

    hv_new, tsrc, tdst = _t0a(nf, pnWT, pnb, ae1T, Wq1, bq1)
    b1 = _t0b(ef, WeT, b1p)
    gs = _gather_rows(tsrc, src, DP)
    gd = _gather_rows(tdst, dst, DQ)
    P = _te1(gs, b1, gd, vp)
    acc = _scatter_add_rows(P, dst, zeros)
    node, tsrc, tdst = _tn1(acc, hv_new, p['ag1_et_W'].T,
                            p['ag1_et_b'][None, :], M1, c1, Wq_l0, bq_l0, gw1)

    gs = _gather_rows(tsrc, src, DP)
    gd = _gather_rows(tdst, dst, DQ)
    P = _te2(gs, gd)
    acc = _scatter_add_rows(P, dst, zeros)
    node, tsrc, tdst = _tn2a(acc, node, M2, c2, Wq_l1, bq_l1, gw_l0)

    gs = _gather_rows(tsrc, src, DP)
    gd = _gather_rows(tdst, dst, DQ)
    P = _te2(gs, gd)
    acc = _scatter_add_rows(P, dst, zeros)
    node = _tn2b(acc, node, gw_l1)

    pflat = []
    for rp in p['readout']:
        R1mat, clb = _qcol(rp['cl_W'][0, :G], rp['cl_b'][0])
        Zw, _ = _qcol(rp['cl_W'][0, G:], 0.0)
        pflat += [R1mat, clb, Zw, rp['prn_W'].T, rp['prn_b'][None, :]]
        gww = _gru_prep(rp['Wih'], rp['Whh'], rp['bih'], rp['bhh'])
        pflat += [gww[k] for k in _GRU_KEYS]
    predWT = jnp.zeros((G, 128), f32).at[:, 0].set(p['pred_W'][0])
    predb = jnp.zeros((1, 128), f32).at[0, 0].set(p['pred_b'][0])
    out = _tr(node, gid.astype(jnp.int32)[:, None], predWT, predb, pflat)
    return out[:, 0:1]


def kernel(node_feats1, node_feats2, node_feats3, edge_feats1, edge_feats2,
           edge_feats3, edge_index1, edge_index2, edge_index3,
           node_graph_ids1, node_graph_ids2, node_graph_ids3,
           params1, params2, params3):
    zeros = jnp.zeros((CPT * NACC,), f32)
    o1 = _tower(params1, node_feats1, edge_feats1, edge_index1,
                node_graph_ids1, zeros)
    o2 = _tower(params2, node_feats2, edge_feats2, edge_index2,
                node_graph_ids2, zeros)
    o3 = _tower(params3, node_feats3, edge_feats3, edge_index3,
                node_graph_ids3, zeros)
    return jnp.concatenate([o1, o2, o3], axis=1)


# speedup vs baseline: 5.2127x; 1.0948x over previous
import functools

import jax
import jax.numpy as jnp
from jax import lax
from jax.experimental import pallas as pl
from jax.experimental.pallas import tpu as pltpu, tpu_sc as plsc

N = 10000
E = 160000
G = 200
DP = 256
DQ = 128
DH = 128
NACC = 10240
NG = 64
BLKN = 1000
BLKE = 6400
NC, NS = 2, 16
NW = NC * NS
CH = 128

f32 = jnp.float32


def _lrelu(x):
    return jnp.where(x > 0, x, 0.01 * x)


def _elu(x):
    return jnp.where(x > 0, x, jnp.exp(x) - 1.0)


@functools.lru_cache(maxsize=None)
def _sc_gather_kernel(D, B):
    n_chunks = B // CH
    n_pairs = (n_chunks + 2 * NW - 1) // (2 * NW)
    mesh = plsc.VectorSubcoreMesh(core_axis_name="c", subcore_axis_name="s")

    @functools.partial(
        pl.kernel, mesh=mesh,
        out_type=jax.ShapeDtypeStruct((B, D), f32),
        scratch_types=[
            pltpu.VMEM((2, CH), jnp.int32),
            pltpu.VMEM((2, CH, D), f32),
            pltpu.SemaphoreType.DMA,
            pltpu.SemaphoreType.DMA,
            pltpu.SemaphoreType.DMA,
            pltpu.SemaphoreType.DMA,
            pltpu.SemaphoreType.DMA,
        ],
    )
    def k(table_hbm, idx_hbm, out_hbm, idx_v, rows_v, gsem, si0, si1, so0, so1):
        c = lax.axis_index("c")
        s = lax.axis_index("s")
        wid = s * NC + c
        sis = (si0, si1)
        sos = (so0, so1)

        def cid_of(i, b):
            return (2 * i + b) * NW + wid

        def start_idx(cid, b):
            @pl.when(cid < n_chunks)
            def _():
                pltpu.async_copy(idx_hbm.at[pl.ds(cid * CH, CH)],
                                 idx_v.at[b], sis[b])

        start_idx(cid_of(0, 0), 0)
        start_idx(cid_of(0, 1), 1)

        def body(i, carry):
            for b in range(2):
                cid = cid_of(i, b)

                @pl.when(cid < n_chunks)
                def _():
                    @pl.when(i > 0)
                    def _():
                        prev = cid - 2 * NW
                        pltpu.make_async_copy(
                            rows_v.at[b],
                            out_hbm.at[pl.ds(prev * CH, CH)], sos[b]).wait()
                    pltpu.make_async_copy(idx_hbm.at[pl.ds(cid * CH, CH)],
                                          idx_v.at[b], sis[b]).wait()
                    pltpu.async_copy(table_hbm.at[idx_v.at[b]],
                                     rows_v.at[b], gsem).wait()
                    start_idx(cid + 2 * NW, b)
                    pltpu.async_copy(rows_v.at[b],
                                     out_hbm.at[pl.ds(cid * CH, CH)], sos[b])
            return carry

        lax.fori_loop(0, n_pairs, body, 0)
        for b in range(2):
            first = wid + b * NW

            @pl.when(first < n_chunks)
            def _():
                ilast = (n_chunks - 1 - first) // (2 * NW)
                cid = cid_of(ilast, b)
                pltpu.make_async_copy(
                    rows_v.at[b], out_hbm.at[pl.ds(cid * CH, CH)],
                    sos[b]).wait()

    return k


CHS = 640
CPT = 8


@functools.lru_cache(maxsize=None)
def _sc_scatter_kernel(B):
    n_chunks = B // CHS
    mesh = plsc.VectorSubcoreMesh(core_axis_name="c", subcore_axis_name="s")

    @functools.partial(
        pl.kernel, mesh=mesh,
        compiler_params=pltpu.CompilerParams(needs_layout_passes=False),
        out_type=jax.ShapeDtypeStruct((DP * NACC,), f32),
        scratch_types=[
            pltpu.VMEM((2 * CHS,), jnp.int32),
            pltpu.VMEM((2, CPT, CHS), f32),
            pltpu.VMEM((CPT * NACC,), f32),
            pltpu.SemaphoreType.DMA,
            pltpu.SemaphoreType.DMA,
            pltpu.SemaphoreType.DMA,
            pltpu.SemaphoreType.DMA,
        ],
    )
    def k(rowsT_hbm, idx_hbm, zerosT_hbm, out_hbm, idx_v, buf_v, acc_v,
          si0, si1, sr0, sr1):
        c = lax.axis_index("c")
        s = lax.axis_index("s")
        wid = s * NC + c
        rbase = wid * CPT
        sis = (si0, si1)
        srs = (sr0, sr1)
        pltpu.sync_copy(zerosT_hbm, acc_v)

        def start(cid, b):
            pltpu.async_copy(idx_hbm.at[pl.ds(cid * CHS, CHS)],
                             idx_v.at[pl.ds(b * CHS, CHS)], sis[b])
            pltpu.async_copy(
                rowsT_hbm.at[pl.ds(rbase, CPT), pl.ds(cid * CHS, CHS)],
                buf_v.at[b], srs[b])

        start(0, 0)
        start(1, 1)

        def body(i, carry):
            for b in range(2):
                cid = i * 2 + b
                pltpu.make_async_copy(idx_hbm.at[pl.ds(cid * CHS, CHS)],
                                      idx_v.at[pl.ds(b * CHS, CHS)],
                                      sis[b]).wait()
                pltpu.make_async_copy(
                    rowsT_hbm.at[pl.ds(rbase, CPT), pl.ds(cid * CHS, CHS)],
                    buf_v.at[b], srs[b]).wait()
                def grp(g, cc):
                    dstv = idx_v[pl.ds(b * CHS + g * 16, 16)]
                    for j in range(CPT):
                        vals = buf_v[b, j, pl.ds(g * 16, 16)]
                        plsc.addupdate_scatter(acc_v, [dstv + (j * NACC)], vals)
                    return cc

                lax.fori_loop(0, CHS // 16, grp, 0)

                @pl.when(cid + 2 < n_chunks)
                def _():
                    start(cid + 2, b)
            return carry

        lax.fori_loop(0, n_chunks // 2, body, 0)
        pltpu.sync_copy(acc_v, out_hbm.at[pl.ds(rbase * NACC, CPT * NACC)])

    return k


def _gather_rows(table, idx, D):
    return _sc_gather_kernel(D, idx.shape[0])(table, idx)


def _scatter_add_rows(rowsT, idx, zerosT):
    accT = _sc_scatter_kernel(rowsT.shape[1])(rowsT, idx, zerosT)
    return accT.reshape(DP, NACC)[:, :N].T


def _mm(x, w):
    return jax.lax.dot_general(x, w, (((1,), (0,)), ((), ())),
                               preferred_element_type=f32)


def _mmT0(x, y):
    return jax.lax.dot_general(x, y, (((0,), (0,)), ((), ())),
                               preferred_element_type=f32)


def _gru_body(x, h, w):
    r = jax.nn.sigmoid(_mm(x, w['WrT']) + w['bir'] + _mm(h, w['UrT']) + w['bhr'])
    z = jax.nn.sigmoid(_mm(x, w['WzT']) + w['biz'] + _mm(h, w['UzT']) + w['bhz'])
    n = jnp.tanh(_mm(x, w['WnT']) + w['bin'] + r * (_mm(h, w['UnT']) + w['bhn']))
    return (1.0 - z) * n + z * h


_GRU_KEYS = ('WrT', 'WzT', 'WnT', 'UrT', 'UzT', 'UnT',
             'bir', 'biz', 'bin', 'bhr', 'bhz', 'bhn')


def _gru_prep(Wih, Whh, bih, bhh):
    return {
        'WrT': Wih[0:G].T, 'WzT': Wih[G:2 * G].T, 'WnT': Wih[2 * G:].T,
        'UrT': Whh[0:G].T, 'UzT': Whh[G:2 * G].T, 'UnT': Whh[2 * G:].T,
        'bir': bih[0:G][None, :], 'biz': bih[G:2 * G][None, :],
        'bin': bih[2 * G:][None, :],
        'bhr': bhh[0:G][None, :], 'bhz': bhh[G:2 * G][None, :],
        'bhn': bhh[2 * G:][None, :],
    }


def _row_spec(blk, w):
    return pl.BlockSpec((blk, w), lambda i: (i, 0))


def _full_spec(shape):
    nd = len(shape)
    return pl.BlockSpec(shape, lambda *_: (0,) * nd)


def _t0a_body(nf, pnWT, pnb, ae1T, Wq, bq, hv_o, tsrc_o, tdst_o):
    hv = _lrelu(_mm(nf[...], pnWT[...]) + pnb[...])
    hv_o[...] = hv
    tsrc_o[...] = _mm(nf[...], ae1T[...])
    tdst_o[...] = _mm(hv, Wq[...]) + bq[...]


def _t0a(nf, pnWT, pnb, ae1T, Wq, bq):
    grid = (N // BLKN,)
    return pl.pallas_call(
        _t0a_body,
        grid=grid,
        in_specs=[_row_spec(BLKN, 128), _full_spec((128, G)), _full_spec((1, G)),
                  _full_spec((128, DP)), _full_spec((G, DQ)), _full_spec((1, DQ))],
        out_specs=[_row_spec(BLKN, G), _row_spec(BLKN, DP), _row_spec(BLKN, DQ)],
        out_shape=[jax.ShapeDtypeStruct((N, G), f32),
                   jax.ShapeDtypeStruct((N, DP), f32),
                   jax.ShapeDtypeStruct((N, DQ), f32)],
    )(nf, pnWT, pnb, ae1T, Wq, bq)


def _t0b_body(ef, WeT, b1p, out):
    out[...] = _mm(ef[...], WeT[...]) + b1p[...]


def _t0b(ef, WeT, b1p):
    return pl.pallas_call(
        _t0b_body,
        grid=(E // BLKE,),
        in_specs=[_row_spec(BLKE, 16), _full_spec((16, DP)), _full_spec((1, DP))],
        out_specs=_row_spec(BLKE, DP),
        out_shape=jax.ShapeDtypeStruct((E, DP), f32),
    )(ef, WeT, b1p)


def _te1_body(gs, ef, WeT, b1p, gd, vp, out):
    he1 = _lrelu(gs[...] + _mm(ef[...], WeT[...]) + b1p[...])
    dot = jnp.sum(he1 * vp[...], axis=1, keepdims=True)
    ex = jnp.exp(_lrelu(gd[..., 0:1] + dot))
    col = lax.broadcasted_iota(jnp.int32, he1.shape, 1)
    out[...] = jnp.where(col == G, ex, he1 * ex).T


def _te1(gs, ef, WeT, b1p, gd, vp):
    return pl.pallas_call(
        _te1_body,
        grid=(E // BLKE,),
        in_specs=[_row_spec(BLKE, DP), _row_spec(BLKE, 16),
                  _full_spec((16, DP)), _full_spec((1, DP)),
                  _row_spec(BLKE, DQ), _full_spec((1, DP))],
        out_specs=pl.BlockSpec((DP, BLKE), lambda i: (0, i)),
        out_shape=jax.ShapeDtypeStruct((DP, E), f32),
    )(gs, ef, WeT, b1p, gd, vp)


def _te2_body(gs, gd, out):
    g = gs[...]
    ex = jnp.exp(_lrelu(g[..., G:G + 1] + gd[..., 0:1]))
    col = lax.broadcasted_iota(jnp.int32, g.shape, 1)
    out[...] = jnp.where(col == G, ex, g * ex).T


def _te2(gs, gd):
    return pl.pallas_call(
        _te2_body,
        grid=(E // BLKE,),
        in_specs=[_row_spec(BLKE, DP), _row_spec(BLKE, DQ)],
        out_specs=pl.BlockSpec((DP, BLKE), lambda i: (0, i)),
        out_shape=jax.ShapeDtypeStruct((DP, E), f32),
    )(gs, gd)


def _acc_combine(a):
    s0 = a[..., G:G + 1]
    safe = jnp.where(s0 > 0, s0, 1.0)
    return s0, a[..., 0:G] / safe


def _tn1_body(a0, hv, etWT, etb, M1, c1, Wq, bq, *gw_refs):
    w = {k: r[...] for k, r in zip(_GRU_KEYS, gw_refs[:12])}
    s0, mean = _acc_combine(a0[...])
    ctx = _elu(jnp.where(s0 > 0, _mm(mean, etWT[...]) + etb[...], 0.0))
    node = jax.nn.relu(_gru_body(ctx, hv[...], w))
    gw_refs[12][...] = node
    gw_refs[13][...] = _mm(node, M1[...]) + c1[...]
    gw_refs[14][...] = _mm(node, Wq[...]) + bq[...]


def _tn1(a0, hv, etWT, etb, M1, c1, Wq, bq, gw):
    ins = [a0, hv, etWT, etb, M1, c1, Wq, bq] + [gw[k] for k in _GRU_KEYS]
    in_specs = ([_row_spec(BLKN, DP), _row_spec(BLKN, G),
                 _full_spec((G, G)), _full_spec((1, G)),
                 _full_spec((G, DP)), _full_spec((1, DP)),
                 _full_spec((G, DQ)), _full_spec((1, DQ))]
                + [_full_spec(gw[k].shape) for k in _GRU_KEYS])
    return pl.pallas_call(
        _tn1_body,
        grid=(N // BLKN,),
        in_specs=in_specs,
        out_specs=[_row_spec(BLKN, G), _row_spec(BLKN, DP), _row_spec(BLKN, DQ)],
        out_shape=[jax.ShapeDtypeStruct((N, G), f32),
                   jax.ShapeDtypeStruct((N, DP), f32),
                   jax.ShapeDtypeStruct((N, DQ), f32)],
    )(*ins)


def _tn2a_body(a0, hprev, M1, c1, Wq, bq, *gw_refs):
    w = {k: r[...] for k, r in zip(_GRU_KEYS, gw_refs[:12])}
    s0, mean = _acc_combine(a0[...])
    cin = _elu(jnp.where(s0 > 0, mean, 0.0))
    node = jax.nn.relu(_gru_body(cin, hprev[...], w))
    gw_refs[12][...] = node
    gw_refs[13][...] = _mm(node, M1[...]) + c1[...]
    gw_refs[14][...] = _mm(node, Wq[...]) + bq[...]


def _tn2a(a0, hprev, M1, c1, Wq, bq, gw):
    ins = [a0, hprev, M1, c1, Wq, bq] + [gw[k] for k in _GRU_KEYS]
    in_specs = ([_row_spec(BLKN, DP), _row_spec(BLKN, G),
                 _full_spec((G, DP)), _full_spec((1, DP)),
                 _full_spec((G, DQ)), _full_spec((1, DQ))]
                + [_full_spec(gw[k].shape) for k in _GRU_KEYS])
    return pl.pallas_call(
        _tn2a_body,
        grid=(N // BLKN,),
        in_specs=in_specs,
        out_specs=[_row_spec(BLKN, G), _row_spec(BLKN, DP), _row_spec(BLKN, DQ)],
        out_shape=[jax.ShapeDtypeStruct((N, G), f32),
                   jax.ShapeDtypeStruct((N, DP), f32),
                   jax.ShapeDtypeStruct((N, DQ), f32)],
    )(*ins)


def _tn2b_body(a0, hprev, *gw_refs):
    w = {k: r[...] for k, r in zip(_GRU_KEYS, gw_refs[:12])}
    s0, mean = _acc_combine(a0[...])
    cin = _elu(jnp.where(s0 > 0, mean, 0.0))
    gw_refs[12][...] = jax.nn.relu(_gru_body(cin, hprev[...], w))


def _tn2b(a0, hprev, gw):
    ins = [a0, hprev] + [gw[k] for k in _GRU_KEYS]
    in_specs = ([_row_spec(BLKN, DP), _row_spec(BLKN, G)]
                + [_full_spec(gw[k].shape) for k in _GRU_KEYS])
    return pl.pallas_call(
        _tn2b_body,
        grid=(N // BLKN,),
        in_specs=in_specs,
        out_specs=_row_spec(BLKN, G),
        out_shape=jax.ShapeDtypeStruct((N, G), f32),
    )(*ins)


def _tr_body(node_r, gid_r, predWT, predb, *prefs):
    node = node_r[...]
    gid = gid_r[...]
    oh = (gid == lax.broadcasted_iota(jnp.int32, (N, NG), 1)).astype(f32)
    g = _mmT0(oh, node)
    for t in range(2):
        base = t * 17
        R1mat, clb, Zw, prnWT, prnb = [r[...] for r in prefs[base:base + 5]]
        w = {k: r[...] for k, r in zip(_GRU_KEYS, prefs[base + 5:base + 17])}
        r1 = _mm(jax.nn.relu(g), R1mat) + clb
        zn = _mm(node, Zw)
        z = _lrelu(_mm(oh, r1) + zn)
        ex = jnp.exp(z - jnp.max(z))
        exv = ex[..., 0:1]
        s = _mmT0(oh, exv)
        hv = _mm(node, prnWT) + prnb
        num = _mmT0(oh, exv * hv)
        grepr = jnp.where(s > 0, num / jnp.where(s > 0, s, 1.0), 0.0)
        g = jax.nn.relu(_gru_body(_elu(grepr), g, w))
    prefs[34][...] = _mm(g, predWT[...]) + predb[...]


def _tr(node, gid2d, predWT, predb, pflat):
    ins = [node, gid2d, predWT, predb] + pflat
    in_specs = [_full_spec(x.shape) for x in ins]
    return pl.pallas_call(
        _tr_body,
        in_specs=in_specs,
        out_specs=_full_spec((NG, 128)),
        out_shape=jax.ShapeDtypeStruct((NG, 128), f32),
    )(*ins)


def _pad_cols(x, w):
    return jnp.pad(x, ((0, 0), (0, w - x.shape[1])))


def _qcol(vec, bias):
    Wq = jnp.zeros((G, DQ), f32).at[:, 0].set(vec)
    bq = jnp.zeros((1, DQ), f32).at[0, 0].set(bias)
    return Wq, bq


def _tower(p, nf, ef, ei, gid, zeros):
    src = ei[0]
    dst = ei[1]

    pnWT = p['pn_W'].T
    pnb = p['pn_b'][None, :]
    ae1T = _pad_cols(p['pe1_W'][:, :128].T, DP)
    Wq1, bq1 = _qcol(p['pe2_W'][0, :G], p['pe2_b'][0])
    WeT = _pad_cols(p['pe1_W'][:, 128:].T, DP)
    b1p = _pad_cols(p['pe1_b'][None, :], DP)
    vp = _pad_cols(p['pe2_W'][0, G:][None, :], DP)
    gw1 = _gru_prep(p['ag1_Wih'], p['ag1_Whh'], p['ag1_bih'], p['ag1_bhh'])

    def layer_tabs(lp):
        M = jnp.concatenate(
            [lp['pn_W'].T, lp['pe_W'][0, G:][:, None],
             jnp.zeros((G, DP - G - 1), f32)], axis=1)
        c = jnp.concatenate(
            [lp['pn_b'], jnp.zeros((DP - G,), f32)])[None, :]
        Wq, bq = _qcol(lp['pe_W'][0, :G], lp['pe_b'][0])
        return M, c, Wq, bq

    M1, c1, Wq_l0, bq_l0 = layer_tabs(p['layers'][0])
    M2, c2, Wq_l1, bq_l1 = layer_tabs(p['layers'][1])
    gw_l0 = _gru_prep(*[p['layers'][0][k] for k in ('Wih', 'Whh', 'bih', 'bhh')])
    gw_l1 = _gru_prep(*[p['layers'][1][k] for k in ('Wih', 'Whh', 'bih', 'bhh')])

    hv_new, tsrc, tdst = _t0a(nf, pnWT, pnb, ae1T, Wq1, bq1)
    gs = _gather_rows(tsrc, src, DP)
    gd = _gather_rows(tdst, dst, DQ)
    P = _te1(gs, ef, WeT, b1p, gd, vp)
    acc = _scatter_add_rows(P, dst, zeros)
    node, tsrc, tdst = _tn1(acc, hv_new, p['ag1_et_W'].T,
                            p['ag1_et_b'][None, :], M1, c1, Wq_l0, bq_l0, gw1)

    gs = _gather_rows(tsrc, src, DP)
    gd = _gather_rows(tdst, dst, DQ)
    P = _te2(gs, gd)
    acc = _scatter_add_rows(P, dst, zeros)
    node, tsrc, tdst = _tn2a(acc, node, M2, c2, Wq_l1, bq_l1, gw_l0)

    gs = _gather_rows(tsrc, src, DP)
    gd = _gather_rows(tdst, dst, DQ)
    P = _te2(gs, gd)
    acc = _scatter_add_rows(P, dst, zeros)
    node = _tn2b(acc, node, gw_l1)

    pflat = []
    for rp in p['readout']:
        R1mat, clb = _qcol(rp['cl_W'][0, :G], rp['cl_b'][0])
        Zw, _ = _qcol(rp['cl_W'][0, G:], 0.0)
        pflat += [R1mat, clb, Zw, rp['prn_W'].T, rp['prn_b'][None, :]]
        gww = _gru_prep(rp['Wih'], rp['Whh'], rp['bih'], rp['bhh'])
        pflat += [gww[k] for k in _GRU_KEYS]
    predWT = jnp.zeros((G, 128), f32).at[:, 0].set(p['pred_W'][0])
    predb = jnp.zeros((1, 128), f32).at[0, 0].set(p['pred_b'][0])
    out = _tr(node, gid.astype(jnp.int32)[:, None], predWT, predb, pflat)
    return out[:, 0:1]


def kernel(node_feats1, node_feats2, node_feats3, edge_feats1, edge_feats2,
           edge_feats3, edge_index1, edge_index2, edge_index3,
           node_graph_ids1, node_graph_ids2, node_graph_ids3,
           params1, params2, params3):
    zeros = jnp.zeros((CPT * NACC,), f32)
    o1 = _tower(params1, node_feats1, edge_feats1, edge_index1,
                node_graph_ids1, zeros)
    o2 = _tower(params2, node_feats2, edge_feats2, edge_index2,
                node_graph_ids2, zeros)
    o3 = _tower(params3, node_feats3, edge_feats3, edge_index3,
                node_graph_ids3, zeros)
    return jnp.concatenate([o1, o2, o3], axis=1)


# speedup vs baseline: 5.2951x; 1.0158x over previous
import functools

import jax
import jax.numpy as jnp
from jax import lax
from jax.experimental import pallas as pl
from jax.experimental.pallas import tpu as pltpu, tpu_sc as plsc

N = 10000
E = 160000
G = 200
DP = 256
DQ = 128
DH = 128
NACC = 10240
NG = 64
BLKN = 1000
BLKE = 6400
NC, NS = 2, 16
NW = NC * NS
CH = 128

f32 = jnp.float32


def _lrelu(x):
    return jnp.where(x > 0, x, 0.01 * x)


def _elu(x):
    return jnp.where(x > 0, x, jnp.exp(x) - 1.0)


@functools.lru_cache(maxsize=None)
def _sc_gather_kernel(D, B):
    n_chunks = B // CH
    n_pairs = (n_chunks + 2 * NW - 1) // (2 * NW)
    mesh = plsc.VectorSubcoreMesh(core_axis_name="c", subcore_axis_name="s")

    @functools.partial(
        pl.kernel, mesh=mesh,
        out_type=jax.ShapeDtypeStruct((B, D), f32),
        scratch_types=[
            pltpu.VMEM((2, CH), jnp.int32),
            pltpu.VMEM((2, CH, D), f32),
            pltpu.SemaphoreType.DMA,
            pltpu.SemaphoreType.DMA,
            pltpu.SemaphoreType.DMA,
            pltpu.SemaphoreType.DMA,
            pltpu.SemaphoreType.DMA,
            pltpu.SemaphoreType.DMA,
        ],
    )
    def k(table_hbm, idx_hbm, out_hbm, idx_v, rows_v, gs0, gs1,
          si0, si1, so0, so1):
        c = lax.axis_index("c")
        s = lax.axis_index("s")
        wid = s * NC + c
        sis = (si0, si1)
        sos = (so0, so1)
        gss = (gs0, gs1)

        def cid_of(i, b):
            return (2 * i + b) * NW + wid

        def start_idx(cid, b):
            @pl.when(cid < n_chunks)
            def _():
                pltpu.async_copy(idx_hbm.at[pl.ds(cid * CH, CH)],
                                 idx_v.at[b], sis[b])

        def finish(pcid, o):
            pltpu.make_async_copy(table_hbm.at[idx_v.at[o]],
                                  rows_v.at[o], gss[o]).wait()
            pltpu.async_copy(rows_v.at[o],
                             out_hbm.at[pl.ds(pcid * CH, CH)], sos[o])
            start_idx(pcid + 2 * NW, o)

        start_idx(cid_of(0, 0), 0)
        start_idx(cid_of(0, 1), 1)

        def body(i, carry):
            for b in range(2):
                cid = cid_of(i, b)

                @pl.when(cid < n_chunks)
                def _():
                    @pl.when(i > 0)
                    def _():
                        prev = cid - 2 * NW
                        pltpu.make_async_copy(
                            rows_v.at[b],
                            out_hbm.at[pl.ds(prev * CH, CH)], sos[b]).wait()
                    pltpu.make_async_copy(idx_hbm.at[pl.ds(cid * CH, CH)],
                                          idx_v.at[b], sis[b]).wait()
                    pltpu.async_copy(table_hbm.at[idx_v.at[b]],
                                     rows_v.at[b], gss[b])
                    pcid = cid - NW

                    @pl.when(pcid >= 0)
                    def _():
                        finish(pcid, 1 - b)
            return carry

        lax.fori_loop(0, n_pairs, body, 0)
        for b in range(2):
            first = wid + b * NW

            @pl.when(first < n_chunks)
            def _():
                ilast = (n_chunks - 1 - first) // (2 * NW)
                cid = cid_of(ilast, b)

                @pl.when(cid + NW >= n_chunks)
                def _():
                    pltpu.make_async_copy(table_hbm.at[idx_v.at[b]],
                                          rows_v.at[b], gss[b]).wait()
                    pltpu.async_copy(rows_v.at[b],
                                     out_hbm.at[pl.ds(cid * CH, CH)], sos[b])
                pltpu.make_async_copy(
                    rows_v.at[b], out_hbm.at[pl.ds(cid * CH, CH)],
                    sos[b]).wait()

    return k


CHS = 640
CPT = 8


@functools.lru_cache(maxsize=None)
def _sc_scatter_kernel(B):
    n_chunks = B // CHS
    mesh = plsc.VectorSubcoreMesh(core_axis_name="c", subcore_axis_name="s")

    @functools.partial(
        pl.kernel, mesh=mesh,
        compiler_params=pltpu.CompilerParams(needs_layout_passes=False),
        out_type=jax.ShapeDtypeStruct((DP * NACC,), f32),
        scratch_types=[
            pltpu.VMEM((2 * CHS,), jnp.int32),
            pltpu.VMEM((2, CPT, CHS), f32),
            pltpu.VMEM((CPT * NACC,), f32),
            pltpu.SemaphoreType.DMA,
            pltpu.SemaphoreType.DMA,
            pltpu.SemaphoreType.DMA,
            pltpu.SemaphoreType.DMA,
        ],
    )
    def k(rowsT_hbm, idx_hbm, zerosT_hbm, out_hbm, idx_v, buf_v, acc_v,
          si0, si1, sr0, sr1):
        c = lax.axis_index("c")
        s = lax.axis_index("s")
        wid = s * NC + c
        rbase = wid * CPT
        sis = (si0, si1)
        srs = (sr0, sr1)
        pltpu.sync_copy(zerosT_hbm, acc_v)

        def start(cid, b):
            pltpu.async_copy(idx_hbm.at[pl.ds(cid * CHS, CHS)],
                             idx_v.at[pl.ds(b * CHS, CHS)], sis[b])
            pltpu.async_copy(
                rowsT_hbm.at[pl.ds(rbase, CPT), pl.ds(cid * CHS, CHS)],
                buf_v.at[b], srs[b])

        start(0, 0)
        start(1, 1)

        def body(i, carry):
            for b in range(2):
                cid = i * 2 + b
                pltpu.make_async_copy(idx_hbm.at[pl.ds(cid * CHS, CHS)],
                                      idx_v.at[pl.ds(b * CHS, CHS)],
                                      sis[b]).wait()
                pltpu.make_async_copy(
                    rowsT_hbm.at[pl.ds(rbase, CPT), pl.ds(cid * CHS, CHS)],
                    buf_v.at[b], srs[b]).wait()
                def grp(g, cc):
                    dstv = idx_v[pl.ds(b * CHS + g * 16, 16)]
                    for j in range(CPT):
                        vals = buf_v[b, j, pl.ds(g * 16, 16)]
                        plsc.addupdate_scatter(acc_v, [dstv + (j * NACC)], vals)
                    return cc

                lax.fori_loop(0, CHS // 16, grp, 0)

                @pl.when(cid + 2 < n_chunks)
                def _():
                    start(cid + 2, b)
            return carry

        lax.fori_loop(0, n_chunks // 2, body, 0)
        pltpu.sync_copy(acc_v, out_hbm.at[pl.ds(rbase * NACC, CPT * NACC)])

    return k


def _gather_rows(table, idx, D):
    return _sc_gather_kernel(D, idx.shape[0])(table, idx)


def _scatter_add_rows(rowsT, idx, zerosT):
    accT = _sc_scatter_kernel(rowsT.shape[1])(rowsT, idx, zerosT)
    return accT.reshape(DP, NACC)[:, :N].T


def _mm(x, w):
    return jax.lax.dot_general(x, w, (((1,), (0,)), ((), ())),
                               preferred_element_type=f32)


def _mmT0(x, y):
    return jax.lax.dot_general(x, y, (((0,), (0,)), ((), ())),
                               preferred_element_type=f32)


def _gru_body(x, h, w):
    r = jax.nn.sigmoid(_mm(x, w['WrT']) + w['bir'] + _mm(h, w['UrT']) + w['bhr'])
    z = jax.nn.sigmoid(_mm(x, w['WzT']) + w['biz'] + _mm(h, w['UzT']) + w['bhz'])
    n = jnp.tanh(_mm(x, w['WnT']) + w['bin'] + r * (_mm(h, w['UnT']) + w['bhn']))
    return (1.0 - z) * n + z * h


_GRU_KEYS = ('WrT', 'WzT', 'WnT', 'UrT', 'UzT', 'UnT',
             'bir', 'biz', 'bin', 'bhr', 'bhz', 'bhn')


def _gru_prep(Wih, Whh, bih, bhh):
    return {
        'WrT': Wih[0:G].T, 'WzT': Wih[G:2 * G].T, 'WnT': Wih[2 * G:].T,
        'UrT': Whh[0:G].T, 'UzT': Whh[G:2 * G].T, 'UnT': Whh[2 * G:].T,
        'bir': bih[0:G][None, :], 'biz': bih[G:2 * G][None, :],
        'bin': bih[2 * G:][None, :],
        'bhr': bhh[0:G][None, :], 'bhz': bhh[G:2 * G][None, :],
        'bhn': bhh[2 * G:][None, :],
    }


def _row_spec(blk, w):
    return pl.BlockSpec((blk, w), lambda i: (i, 0))


def _full_spec(shape):
    nd = len(shape)
    return pl.BlockSpec(shape, lambda *_: (0,) * nd)


def _t0a_body(nf, pnWT, pnb, ae1T, Wq, bq, hv_o, tsrc_o, tdst_o):
    hv = _lrelu(_mm(nf[...], pnWT[...]) + pnb[...])
    hv_o[...] = hv
    tsrc_o[...] = _mm(nf[...], ae1T[...])
    tdst_o[...] = _mm(hv, Wq[...]) + bq[...]


def _t0a(nf, pnWT, pnb, ae1T, Wq, bq):
    grid = (N // BLKN,)
    return pl.pallas_call(
        _t0a_body,
        grid=grid,
        in_specs=[_row_spec(BLKN, 128), _full_spec((128, G)), _full_spec((1, G)),
                  _full_spec((128, DP)), _full_spec((G, DQ)), _full_spec((1, DQ))],
        out_specs=[_row_spec(BLKN, G), _row_spec(BLKN, DP), _row_spec(BLKN, DQ)],
        out_shape=[jax.ShapeDtypeStruct((N, G), f32),
                   jax.ShapeDtypeStruct((N, DP), f32),
                   jax.ShapeDtypeStruct((N, DQ), f32)],
    )(nf, pnWT, pnb, ae1T, Wq, bq)


def _t0b_body(ef, WeT, b1p, out):
    out[...] = _mm(ef[...], WeT[...]) + b1p[...]


def _t0b(ef, WeT, b1p):
    return pl.pallas_call(
        _t0b_body,
        grid=(E // BLKE,),
        in_specs=[_row_spec(BLKE, 16), _full_spec((16, DP)), _full_spec((1, DP))],
        out_specs=_row_spec(BLKE, DP),
        out_shape=jax.ShapeDtypeStruct((E, DP), f32),
    )(ef, WeT, b1p)


def _te1_body(gs, ef, WeT, b1p, gd, vp, out):
    he1 = _lrelu(gs[...] + _mm(ef[...], WeT[...]) + b1p[...])
    dot = jnp.sum(he1 * vp[...], axis=1, keepdims=True)
    ex = jnp.exp(_lrelu(gd[..., 0:1] + dot))
    col = lax.broadcasted_iota(jnp.int32, he1.shape, 1)
    out[...] = jnp.where(col == G, ex, he1 * ex).T


def _te1(gs, ef, WeT, b1p, gd, vp):
    return pl.pallas_call(
        _te1_body,
        grid=(E // BLKE,),
        in_specs=[_row_spec(BLKE, DP), _row_spec(BLKE, 16),
                  _full_spec((16, DP)), _full_spec((1, DP)),
                  _row_spec(BLKE, DQ), _full_spec((1, DP))],
        out_specs=pl.BlockSpec((DP, BLKE), lambda i: (0, i)),
        out_shape=jax.ShapeDtypeStruct((DP, E), f32),
    )(gs, ef, WeT, b1p, gd, vp)


def _te2_body(gs, gd, out):
    g = gs[...]
    ex = jnp.exp(_lrelu(g[..., G:G + 1] + gd[..., 0:1]))
    col = lax.broadcasted_iota(jnp.int32, g.shape, 1)
    out[...] = jnp.where(col == G, ex, g * ex).T


def _te2(gs, gd):
    return pl.pallas_call(
        _te2_body,
        grid=(E // BLKE,),
        in_specs=[_row_spec(BLKE, DP), _row_spec(BLKE, DQ)],
        out_specs=pl.BlockSpec((DP, BLKE), lambda i: (0, i)),
        out_shape=jax.ShapeDtypeStruct((DP, E), f32),
    )(gs, gd)


def _acc_combine(a):
    s0 = a[..., G:G + 1]
    safe = jnp.where(s0 > 0, s0, 1.0)
    return s0, a[..., 0:G] / safe


def _tn1_body(a0, hv, etWT, etb, M1, c1, Wq, bq, *gw_refs):
    w = {k: r[...] for k, r in zip(_GRU_KEYS, gw_refs[:12])}
    s0, mean = _acc_combine(a0[...])
    ctx = _elu(jnp.where(s0 > 0, _mm(mean, etWT[...]) + etb[...], 0.0))
    node = jax.nn.relu(_gru_body(ctx, hv[...], w))
    gw_refs[12][...] = node
    gw_refs[13][...] = _mm(node, M1[...]) + c1[...]
    gw_refs[14][...] = _mm(node, Wq[...]) + bq[...]


def _tn1(a0, hv, etWT, etb, M1, c1, Wq, bq, gw):
    ins = [a0, hv, etWT, etb, M1, c1, Wq, bq] + [gw[k] for k in _GRU_KEYS]
    in_specs = ([_row_spec(BLKN, DP), _row_spec(BLKN, G),
                 _full_spec((G, G)), _full_spec((1, G)),
                 _full_spec((G, DP)), _full_spec((1, DP)),
                 _full_spec((G, DQ)), _full_spec((1, DQ))]
                + [_full_spec(gw[k].shape) for k in _GRU_KEYS])
    return pl.pallas_call(
        _tn1_body,
        grid=(N // BLKN,),
        in_specs=in_specs,
        out_specs=[_row_spec(BLKN, G), _row_spec(BLKN, DP), _row_spec(BLKN, DQ)],
        out_shape=[jax.ShapeDtypeStruct((N, G), f32),
                   jax.ShapeDtypeStruct((N, DP), f32),
                   jax.ShapeDtypeStruct((N, DQ), f32)],
    )(*ins)


def _tn2a_body(a0, hprev, M1, c1, Wq, bq, *gw_refs):
    w = {k: r[...] for k, r in zip(_GRU_KEYS, gw_refs[:12])}
    s0, mean = _acc_combine(a0[...])
    cin = _elu(jnp.where(s0 > 0, mean, 0.0))
    node = jax.nn.relu(_gru_body(cin, hprev[...], w))
    gw_refs[12][...] = node
    gw_refs[13][...] = _mm(node, M1[...]) + c1[...]
    gw_refs[14][...] = _mm(node, Wq[...]) + bq[...]


def _tn2a(a0, hprev, M1, c1, Wq, bq, gw):
    ins = [a0, hprev, M1, c1, Wq, bq] + [gw[k] for k in _GRU_KEYS]
    in_specs = ([_row_spec(BLKN, DP), _row_spec(BLKN, G),
                 _full_spec((G, DP)), _full_spec((1, DP)),
                 _full_spec((G, DQ)), _full_spec((1, DQ))]
                + [_full_spec(gw[k].shape) for k in _GRU_KEYS])
    return pl.pallas_call(
        _tn2a_body,
        grid=(N // BLKN,),
        in_specs=in_specs,
        out_specs=[_row_spec(BLKN, G), _row_spec(BLKN, DP), _row_spec(BLKN, DQ)],
        out_shape=[jax.ShapeDtypeStruct((N, G), f32),
                   jax.ShapeDtypeStruct((N, DP), f32),
                   jax.ShapeDtypeStruct((N, DQ), f32)],
    )(*ins)


def _tn2b_body(a0, hprev, *gw_refs):
    w = {k: r[...] for k, r in zip(_GRU_KEYS, gw_refs[:12])}
    s0, mean = _acc_combine(a0[...])
    cin = _elu(jnp.where(s0 > 0, mean, 0.0))
    gw_refs[12][...] = jax.nn.relu(_gru_body(cin, hprev[...], w))


def _tn2b(a0, hprev, gw):
    ins = [a0, hprev] + [gw[k] for k in _GRU_KEYS]
    in_specs = ([_row_spec(BLKN, DP), _row_spec(BLKN, G)]
                + [_full_spec(gw[k].shape) for k in _GRU_KEYS])
    return pl.pallas_call(
        _tn2b_body,
        grid=(N // BLKN,),
        in_specs=in_specs,
        out_specs=_row_spec(BLKN, G),
        out_shape=jax.ShapeDtypeStruct((N, G), f32),
    )(*ins)


def _tr_body(node_r, gid_r, predWT, predb, *prefs):
    node = node_r[...]
    gid = gid_r[...]
    oh = (gid == lax.broadcasted_iota(jnp.int32, (N, NG), 1)).astype(f32)
    g = _mmT0(oh, node)
    for t in range(2):
        base = t * 17
        R1mat, clb, Zw, prnWT, prnb = [r[...] for r in prefs[base:base + 5]]
        w = {k: r[...] for k, r in zip(_GRU_KEYS, prefs[base + 5:base + 17])}
        r1 = _mm(jax.nn.relu(g), R1mat) + clb
        zn = _mm(node, Zw)
        z = _lrelu(_mm(oh, r1) + zn)
        ex = jnp.exp(z - jnp.max(z))
        exv = ex[..., 0:1]
        s = _mmT0(oh, exv)
        hv = _mm(node, prnWT) + prnb
        num = _mmT0(oh, exv * hv)
        grepr = jnp.where(s > 0, num / jnp.where(s > 0, s, 1.0), 0.0)
        g = jax.nn.relu(_gru_body(_elu(grepr), g, w))
    prefs[34][...] = _mm(g, predWT[...]) + predb[...]


def _tr(node, gid2d, predWT, predb, pflat):
    ins = [node, gid2d, predWT, predb] + pflat
    in_specs = [_full_spec(x.shape) for x in ins]
    return pl.pallas_call(
        _tr_body,
        in_specs=in_specs,
        out_specs=_full_spec((NG, 128)),
        out_shape=jax.ShapeDtypeStruct((NG, 128), f32),
    )(*ins)


def _pad_cols(x, w):
    return jnp.pad(x, ((0, 0), (0, w - x.shape[1])))


def _qcol(vec, bias):
    Wq = jnp.zeros((G, DQ), f32).at[:, 0].set(vec)
    bq = jnp.zeros((1, DQ), f32).at[0, 0].set(bias)
    return Wq, bq


def _tower(p, nf, ef, ei, gid, zeros):
    src = ei[0]
    dst = ei[1]

    pnWT = p['pn_W'].T
    pnb = p['pn_b'][None, :]
    ae1T = _pad_cols(p['pe1_W'][:, :128].T, DP)
    Wq1, bq1 = _qcol(p['pe2_W'][0, :G], p['pe2_b'][0])
    WeT = _pad_cols(p['pe1_W'][:, 128:].T, DP)
    b1p = _pad_cols(p['pe1_b'][None, :], DP)
    vp = _pad_cols(p['pe2_W'][0, G:][None, :], DP)
    gw1 = _gru_prep(p['ag1_Wih'], p['ag1_Whh'], p['ag1_bih'], p['ag1_bhh'])

    def layer_tabs(lp):
        M = jnp.concatenate(
            [lp['pn_W'].T, lp['pe_W'][0, G:][:, None],
             jnp.zeros((G, DP - G - 1), f32)], axis=1)
        c = jnp.concatenate(
            [lp['pn_b'], jnp.zeros((DP - G,), f32)])[None, :]
        Wq, bq = _qcol(lp['pe_W'][0, :G], lp['pe_b'][0])
        return M, c, Wq, bq

    M1, c1, Wq_l0, bq_l0 = layer_tabs(p['layers'][0])
    M2, c2, Wq_l1, bq_l1 = layer_tabs(p['layers'][1])
    gw_l0 = _gru_prep(*[p['layers'][0][k] for k in ('Wih', 'Whh', 'bih', 'bhh')])
    gw_l1 = _gru_prep(*[p['layers'][1][k] for k in ('Wih', 'Whh', 'bih', 'bhh')])

    hv_new, tsrc, tdst = _t0a(nf, pnWT, pnb, ae1T, Wq1, bq1)
    gs = _gather_rows(tsrc, src, DP)
    gd = _gather_rows(tdst, dst, DQ)
    P = _te1(gs, ef, WeT, b1p, gd, vp)
    acc = _scatter_add_rows(P, dst, zeros)
    node, tsrc, tdst = _tn1(acc, hv_new, p['ag1_et_W'].T,
                            p['ag1_et_b'][None, :], M1, c1, Wq_l0, bq_l0, gw1)

    gs = _gather_rows(tsrc, src, DP)
    gd = _gather_rows(tdst, dst, DQ)
    P = _te2(gs, gd)
    acc = _scatter_add_rows(P, dst, zeros)
    node, tsrc, tdst = _tn2a(acc, node, M2, c2, Wq_l1, bq_l1, gw_l0)

    gs = _gather_rows(tsrc, src, DP)
    gd = _gather_rows(tdst, dst, DQ)
    P = _te2(gs, gd)
    acc = _scatter_add_rows(P, dst, zeros)
    node = _tn2b(acc, node, gw_l1)

    pflat = []
    for rp in p['readout']:
        R1mat, clb = _qcol(rp['cl_W'][0, :G], rp['cl_b'][0])
        Zw, _ = _qcol(rp['cl_W'][0, G:], 0.0)
        pflat += [R1mat, clb, Zw, rp['prn_W'].T, rp['prn_b'][None, :]]
        gww = _gru_prep(rp['Wih'], rp['Whh'], rp['bih'], rp['bhh'])
        pflat += [gww[k] for k in _GRU_KEYS]
    predWT = jnp.zeros((G, 128), f32).at[:, 0].set(p['pred_W'][0])
    predb = jnp.zeros((1, 128), f32).at[0, 0].set(p['pred_b'][0])
    out = _tr(node, gid.astype(jnp.int32)[:, None], predWT, predb, pflat)
    return out[:, 0:1]


def kernel(node_feats1, node_feats2, node_feats3, edge_feats1, edge_feats2,
           edge_feats3, edge_index1, edge_index2, edge_index3,
           node_graph_ids1, node_graph_ids2, node_graph_ids3,
           params1, params2, params3):
    zeros = jnp.zeros((CPT * NACC,), f32)
    o1 = _tower(params1, node_feats1, edge_feats1, edge_index1,
                node_graph_ids1, zeros)
    o2 = _tower(params2, node_feats2, edge_feats2, edge_index2,
                node_graph_ids2, zeros)
    o3 = _tower(params3, node_feats3, edge_feats3, edge_index3,
                node_graph_ids3, zeros)
    return jnp.concatenate([o1, o2, o3], axis=1)
